# baseline (speedup 1.0000x reference)
import functools

import jax
import jax.numpy as jnp
from jax import lax
from jax.experimental import pallas as pl
from jax.experimental.pallas import tpu as pltpu
from jax.experimental.pallas import tpu_sc as plsc

N = 100000
E = 3200000
E_PAD = 3211264
ROWS = E_PAD // 128
ROWS_W = ROWS // 32
BLK_ROWS = 4
N_BLOCKS = ROWS_W // BLK_ROWS
BLKW = 512
ACC_N = N + 160
DEG_N = 100352
STRIPE = ACC_N // 16
OUT_STRIPE = N // 16
ZCH = 313
OCH = 250

_mesh = plsc.VectorSubcoreMesh(core_axis_name="c", subcore_axis_name="s")


def _deg_body(dst2d, part, acc, dstbuf, ones_v, zbuf, sem):
    c = lax.axis_index("c")
    t = lax.axis_index("s")
    wid = c * 16 + t
    for g in range(8):
        ones_v[pl.ds(g * 16, 16)] = jnp.ones((16,), jnp.float32)

    zs = DEG_N // 16

    def zfill(i, carry):
        zbuf[pl.ds(i * 16, 16)] = jnp.zeros((16,), jnp.float32)
        return carry

    lax.fori_loop(0, zs // 16, zfill, 0)
    pltpu.sync_copy(zbuf, acc.at[pl.ds(t * zs, zs)])
    plsc.subcore_barrier()

    def body(b, carry):
        rowbase = wid * ROWS_W + b * BLK_ROWS
        pltpu.sync_copy(dst2d.at[pl.ds(rowbase, BLK_ROWS)], dstbuf)
        for j in range(BLK_ROWS):
            pltpu.sync_copy(ones_v, acc.at[dstbuf.at[j]], add=True)
        return carry

    lax.fori_loop(0, N_BLOCKS, body, 0)
    plsc.subcore_barrier()
    pltpu.sync_copy(acc.at[pl.ds(t * zs, zs)], zbuf)
    pltpu.sync_copy(zbuf, part.at[pl.ds(c * DEG_N + t * zs, zs)])


@functools.partial(
    pl.kernel,
    mesh=_mesh,
    out_type=jax.ShapeDtypeStruct((2 * DEG_N,), jnp.float32),
    scratch_types=[
        pltpu.VMEM_SHARED((DEG_N,), jnp.float32),
        pltpu.VMEM((BLK_ROWS, 128), jnp.int32),
        pltpu.VMEM((128,), jnp.float32),
        pltpu.VMEM((DEG_N // 16,), jnp.float32),
        pltpu.SemaphoreType.DMA,
    ],
)
def _deg_kernel(dst2d, part, acc, dstbuf, ones_v, zbuf, sem):
    _deg_body(dst2d, part, acc, dstbuf, ones_v, zbuf, sem)


def _agg_body(S, P, gtab, idxS, dstf, dummy, out, acc, dstbuf,
              idxbuf, rows_v, zbuf, obuf, sem0, sem1, ssem0, ssem1):
    c = lax.axis_index("c")
    t = lax.axis_index("s")
    edges_t = E_PAD // 16
    nblocks = edges_t // BLKW
    npair = nblocks // 2
    gsems = [sem0, sem1]
    ssems = [ssem0, ssem1]

    def zfill(i, carry):
        zbuf[i, :] = jnp.zeros((16,), jnp.float32)
        return carry

    lax.fori_loop(0, ZCH, zfill, 0)

    def load_and_fire(par, ebase, s):
        pltpu.sync_copy(idxS.at[s, pl.ds(ebase, BLKW)], idxbuf.at[par])
        pltpu.sync_copy(dstf.at[pl.ds(ebase, BLKW)], dstbuf.at[par])
        pltpu.async_copy(gtab.at[idxbuf.at[par]], rows_v.at[par], gsems[par])

    def drain_g(par):
        pltpu.make_async_copy(dummy, rows_v.at[par], gsems[par]).wait()

    def fire_scatter(par):
        pltpu.async_copy(rows_v.at[par], acc.at[dstbuf.at[par]], ssems[par],
                         add=True)

    def drain_s(par):
        pltpu.make_async_copy(dummy, rows_v.at[par], ssems[par]).wait()

    for p in range(P):
        s = c + 2 * p
        def zcopy(i, carry):
            pltpu.sync_copy(zbuf, acc.at[pl.ds(t * STRIPE + i * ZCH, ZCH), :])
            return carry

        lax.fori_loop(0, STRIPE // ZCH, zcopy, 0)
        plsc.subcore_barrier()

        base0 = t * edges_t
        load_and_fire(0, base0, s)

        def body(i, carry):
            base = base0 + 2 * i * BLKW
            drain_g(0)

            @pl.when(i > 0)
            def _():
                drain_s(1)

            load_and_fire(1, base + BLKW, s)
            fire_scatter(0)
            drain_g(1)

            @pl.when(i < npair - 1)
            def _():
                drain_s(0)
                load_and_fire(0, base + 2 * BLKW, s)

            fire_scatter(1)
            return carry

        lax.fori_loop(0, npair, body, 0)
        drain_s(0)
        drain_s(1)
        plsc.subcore_barrier()

        def ocopy(i, carry):
            base = t * OUT_STRIPE + i * OCH
            pltpu.sync_copy(acc.at[pl.ds(base, OCH), :], obuf)
            pltpu.sync_copy(obuf, out.at[pl.ds(base, OCH), pl.ds(16 * s, 16)])
            return carry

        lax.fori_loop(0, OUT_STRIPE // OCH, ocopy, 0)
        plsc.subcore_barrier()


def _make_agg_kernel(S, P):
    @functools.partial(
        pl.kernel,
        mesh=_mesh,
        compiler_params=pltpu.CompilerParams(use_tc_tiling_on_sc=False),
        out_type=jax.ShapeDtypeStruct((N, 16 * S), jnp.float32),
        scratch_types=[
            pltpu.VMEM_SHARED((ACC_N, 16), jnp.float32),
            pltpu.VMEM((2, BLKW), jnp.int32),
            pltpu.VMEM((2, BLKW), jnp.int32),
            pltpu.VMEM((2, BLKW, 16), jnp.float32),
            pltpu.VMEM((ZCH, 16), jnp.float32),
            pltpu.VMEM((OCH, 16), jnp.float32),
            pltpu.SemaphoreType.DMA,
            pltpu.SemaphoreType.DMA,
            pltpu.SemaphoreType.DMA,
            pltpu.SemaphoreType.DMA,
        ],
    )
    def k(gtab, idxS, dstf, dummy, out, acc, dstbuf, idxbuf,
          rows_v, zbuf, obuf, sem0, sem1, ssem0, ssem1):
        _agg_body(S, P, gtab, idxS, dstf, dummy, out, acc,
                  dstbuf, idxbuf, rows_v, zbuf, obuf, sem0, sem1,
                  ssem0, ssem1)

    return k


_agg32 = _make_agg_kernel(2, 1)


def _dinv_kernel(part_ref, o_ref):
    deg = part_ref[0, :] + part_ref[1, :] + 1.0
    o_ref[0, :] = jax.lax.rsqrt(deg)


def _dinv(part):
    return pl.pallas_call(
        _dinv_kernel,
        out_shape=jax.ShapeDtypeStruct((1, DEG_N), jnp.float32),
    )(part)


def _q1_kernel(x_ref, dinv_ref, o_ref):
    xb = x_ref[...] * dinv_ref[...]
    o_ref[...] = jnp.concatenate(
        [xb, jnp.zeros((xb.shape[0], 14), jnp.float32)], axis=1)


def _q1(x, dinv2d):
    blk = 10000
    return pl.pallas_call(
        _q1_kernel,
        grid=(N // blk,),
        in_specs=[
            pl.BlockSpec((blk, 18), lambda i: (i, 0)),
            pl.BlockSpec((blk, 1), lambda i: (i, 0)),
        ],
        out_specs=pl.BlockSpec((blk, 32), lambda i: (i, 0)),
        out_shape=jax.ShapeDtypeStruct((N, 32), jnp.float32),
    )(x, dinv2d)


def _mid_kernel(t_ref, q_ref, dinv_ref, w_ref, b_ref, o_ref):
    m = (t_ref[...] + q_ref[...]) * dinv_ref[...]
    h = jnp.maximum(jnp.dot(m, w_ref[...],
                            preferred_element_type=jnp.float32) + b_ref[...],
                    0.0)
    o_ref[...] = h * dinv_ref[...]


def _mid(t1, q1, dinv2d, W1p, b1):
    blk = 10000
    return pl.pallas_call(
        _mid_kernel,
        grid=(N // blk,),
        in_specs=[
            pl.BlockSpec((blk, 32), lambda i: (i, 0)),
            pl.BlockSpec((blk, 32), lambda i: (i, 0)),
            pl.BlockSpec((blk, 1), lambda i: (i, 0)),
            pl.BlockSpec((32, 32), lambda i: (0, 0)),
            pl.BlockSpec((1, 32), lambda i: (0, 0)),
        ],
        out_specs=pl.BlockSpec((blk, 32), lambda i: (i, 0)),
        out_shape=jax.ShapeDtypeStruct((N, 32), jnp.float32),
    )(t1, q1, dinv2d, W1p, b1.reshape(1, 32))


def _final_kernel(t_ref, q_ref, dinv_ref, w_ref, b_ref, wfc_ref, bfc_ref,
                  o_ref):
    m = (t_ref[...] + q_ref[...]) * dinv_ref[...]
    h = jnp.maximum(jnp.dot(m, w_ref[...],
                            preferred_element_type=jnp.float32) + b_ref[...],
                    0.0)
    logits = jnp.dot(h, wfc_ref[...],
                     preferred_element_type=jnp.float32) + bfc_ref[...]
    mx = jnp.max(logits, axis=1, keepdims=True)
    z = logits - mx
    lse = jnp.log(jnp.sum(jnp.exp(z), axis=1, keepdims=True))
    o_ref[...] = z - lse


def _final(t2, q2, dinv2d, W2, b2, Wfc, bfc):
    blk = 10000
    return pl.pallas_call(
        _final_kernel,
        grid=(N // blk,),
        in_specs=[
            pl.BlockSpec((blk, 32), lambda i: (i, 0)),
            pl.BlockSpec((blk, 32), lambda i: (i, 0)),
            pl.BlockSpec((blk, 1), lambda i: (i, 0)),
            pl.BlockSpec((32, 64), lambda i: (0, 0)),
            pl.BlockSpec((1, 64), lambda i: (0, 0)),
            pl.BlockSpec((64, 2), lambda i: (0, 0)),
            pl.BlockSpec((1, 2), lambda i: (0, 0)),
        ],
        out_specs=pl.BlockSpec((blk, 2), lambda i: (i, 0)),
        out_shape=jax.ShapeDtypeStruct((N, 2), jnp.float32),
    )(t2, q2, dinv2d, W2, b2.reshape(1, 64), Wfc, bfc.reshape(1, 2))


def kernel(x, edge_index, W1, b1, W2, b2, Wfc, bfc):
    src = edge_index[0].astype(jnp.int32)
    dst = edge_index[1].astype(jnp.int32)
    pad = E_PAD - E
    pad_i = jnp.arange(pad, dtype=jnp.int32)
    srcp = jnp.concatenate([src, pad_i % 128])
    dstp = jnp.concatenate([dst, N + (pad_i % 160)])
    dst2d = dstp.reshape(ROWS, 128)
    idx1 = srcp[None, :] * 2 + jnp.arange(2, dtype=jnp.int32)[:, None]
    W1p = jnp.concatenate([W1, jnp.zeros((14, 32), jnp.float32)], axis=0)

    dummy = jnp.zeros((BLKW, 16), jnp.float32)

    part = _deg_kernel(dst2d).reshape(2, DEG_N)
    dinv2d = _dinv(part).reshape(DEG_N, 1)[:N]

    q1 = _q1(x, dinv2d)
    t1 = _agg32(q1.reshape(2 * N, 16), idx1, dstp, dummy)
    q2 = _mid(t1, q1, dinv2d, W1p, b1)
    t2 = _agg32(q2.reshape(2 * N, 16), idx1, dstp, dummy)
    return _final(t2, q2, dinv2d, W2, b2, Wfc, bfc)


# speedup vs baseline: 1.0288x; 1.0288x over previous
import functools

import jax
import jax.numpy as jnp
from jax import lax
from jax.experimental import pallas as pl
from jax.experimental.pallas import tpu as pltpu
from jax.experimental.pallas import tpu_sc as plsc

N = 100000
E = 3200000
E_PAD = 3211264
ROWS = E_PAD // 128
ROWS_W = ROWS // 32
BLK_ROWS = 4
N_BLOCKS = ROWS_W // BLK_ROWS
BLKW = 512
ACC_N = N + 160
DEG_N = 100352
STRIPE = ACC_N // 16
OUT_STRIPE = N // 16
ZCH = 313
OCH = 250

_mesh = plsc.VectorSubcoreMesh(core_axis_name="c", subcore_axis_name="s")


def _deg_body(dstf, dummyd, part, acc, dstbuf0, dstbuf1, ones_v, zbuf,
              sem0, sem1):
    c = lax.axis_index("c")
    t = lax.axis_index("s")
    wid = c * 16 + t
    edges_w = E_PAD // 32
    nblocks = edges_w // BLKW
    npair = nblocks // 2
    ssems = [sem0, sem1]
    dstbufs = [dstbuf0, dstbuf1]

    def ofill(i, carry):
        ones_v[pl.ds(i * 16, 16)] = jnp.ones((16,), jnp.float32)
        return carry

    lax.fori_loop(0, BLKW // 16, ofill, 0)

    zs = DEG_N // 16

    def zfill(i, carry):
        zbuf[pl.ds(i * 16, 16)] = jnp.zeros((16,), jnp.float32)
        return carry

    lax.fori_loop(0, zs // 16, zfill, 0)
    pltpu.sync_copy(zbuf, acc.at[pl.ds(t * zs, zs)])
    plsc.subcore_barrier()

    def load(par, ebase):
        pltpu.sync_copy(dstf.at[pl.ds(ebase, BLKW)], dstbufs[par])

    def fire(par):
        pltpu.async_copy(ones_v, acc.at[dstbufs[par]], ssems[par],
                         add=True)

    def drain(par):
        pltpu.make_async_copy(dummyd, ones_v, ssems[par]).wait()

    base0 = wid * edges_w
    load(0, base0)

    def body(i, carry):
        base = base0 + 2 * i * BLKW
        fire(0)

        @pl.when(i > 0)
        def _():
            drain(1)

        load(1, base + BLKW)
        fire(1)

        @pl.when(i < npair - 1)
        def _():
            drain(0)
            load(0, base + 2 * BLKW)

        return carry

    lax.fori_loop(0, npair, body, 0)
    drain(0)
    drain(1)
    plsc.subcore_barrier()
    pltpu.sync_copy(acc.at[pl.ds(t * zs, zs)], zbuf)
    pltpu.sync_copy(zbuf, part.at[pl.ds(c * DEG_N + t * zs, zs)])


@functools.partial(
    pl.kernel,
    mesh=_mesh,
    out_type=jax.ShapeDtypeStruct((2 * DEG_N,), jnp.float32),
    scratch_types=[
        pltpu.VMEM_SHARED((DEG_N,), jnp.float32),
        pltpu.VMEM((BLKW,), jnp.int32),
        pltpu.VMEM((BLKW,), jnp.int32),
        pltpu.VMEM((BLKW,), jnp.float32),
        pltpu.VMEM((DEG_N // 16,), jnp.float32),
        pltpu.SemaphoreType.DMA,
        pltpu.SemaphoreType.DMA,
    ],
)
def _deg_kernel(dstf, dummyd, part, acc, dstbuf0, dstbuf1, ones_v, zbuf,
                sem0, sem1):
    _deg_body(dstf, dummyd, part, acc, dstbuf0, dstbuf1, ones_v, zbuf,
              sem0, sem1)


def _agg_body(S, P, gtab, idxS, dstf, dummy, out, acc, dstbuf,
              idxbuf, rows_v, zbuf, obuf, sem0, sem1, ssem0, ssem1):
    c = lax.axis_index("c")
    t = lax.axis_index("s")
    edges_t = E_PAD // 16
    nblocks = edges_t // BLKW
    npair = nblocks // 2
    gsems = [sem0, sem1]
    ssems = [ssem0, ssem1]

    def zfill(i, carry):
        zbuf[i, :] = jnp.zeros((16,), jnp.float32)
        return carry

    lax.fori_loop(0, ZCH, zfill, 0)

    def load_and_fire(par, ebase, s):
        pltpu.sync_copy(idxS.at[s, pl.ds(ebase, BLKW)], idxbuf.at[par])
        pltpu.sync_copy(dstf.at[pl.ds(ebase, BLKW)], dstbuf.at[par])
        pltpu.async_copy(gtab.at[idxbuf.at[par]], rows_v.at[par], gsems[par])

    def drain_g(par):
        pltpu.make_async_copy(dummy, rows_v.at[par], gsems[par]).wait()

    def fire_scatter(par):
        pltpu.async_copy(rows_v.at[par], acc.at[dstbuf.at[par]], ssems[par],
                         add=True)

    def drain_s(par):
        pltpu.make_async_copy(dummy, rows_v.at[par], ssems[par]).wait()

    for p in range(P):
        s = c + 2 * p
        def zcopy(i, carry):
            pltpu.sync_copy(zbuf, acc.at[pl.ds(t * STRIPE + i * ZCH, ZCH), :])
            return carry

        lax.fori_loop(0, STRIPE // ZCH, zcopy, 0)
        plsc.subcore_barrier()

        base0 = t * edges_t
        load_and_fire(0, base0, s)

        def body(i, carry):
            base = base0 + 2 * i * BLKW
            drain_g(0)

            @pl.when(i > 0)
            def _():
                drain_s(1)

            load_and_fire(1, base + BLKW, s)
            fire_scatter(0)
            drain_g(1)

            @pl.when(i < npair - 1)
            def _():
                drain_s(0)
                load_and_fire(0, base + 2 * BLKW, s)

            fire_scatter(1)
            return carry

        lax.fori_loop(0, npair, body, 0)
        drain_s(0)
        drain_s(1)
        plsc.subcore_barrier()

        def ocopy(i, carry):
            base = t * OUT_STRIPE + i * OCH
            pltpu.sync_copy(acc.at[pl.ds(base, OCH), :], obuf)
            pltpu.sync_copy(obuf, out.at[pl.ds(base, OCH), pl.ds(16 * s, 16)])
            return carry

        lax.fori_loop(0, OUT_STRIPE // OCH, ocopy, 0)
        plsc.subcore_barrier()


def _make_agg_kernel(S, P):
    @functools.partial(
        pl.kernel,
        mesh=_mesh,
        compiler_params=pltpu.CompilerParams(use_tc_tiling_on_sc=False),
        out_type=jax.ShapeDtypeStruct((N, 16 * S), jnp.float32),
        scratch_types=[
            pltpu.VMEM_SHARED((ACC_N, 16), jnp.float32),
            pltpu.VMEM((2, BLKW), jnp.int32),
            pltpu.VMEM((2, BLKW), jnp.int32),
            pltpu.VMEM((2, BLKW, 16), jnp.float32),
            pltpu.VMEM((ZCH, 16), jnp.float32),
            pltpu.VMEM((OCH, 16), jnp.float32),
            pltpu.SemaphoreType.DMA,
            pltpu.SemaphoreType.DMA,
            pltpu.SemaphoreType.DMA,
            pltpu.SemaphoreType.DMA,
        ],
    )
    def k(gtab, idxS, dstf, dummy, out, acc, dstbuf, idxbuf,
          rows_v, zbuf, obuf, sem0, sem1, ssem0, ssem1):
        _agg_body(S, P, gtab, idxS, dstf, dummy, out, acc,
                  dstbuf, idxbuf, rows_v, zbuf, obuf, sem0, sem1,
                  ssem0, ssem1)

    return k


_agg32 = _make_agg_kernel(2, 1)


def _dinv_kernel(part_ref, o_ref):
    deg = part_ref[0, :] + part_ref[1, :] + 1.0
    o_ref[0, :] = jax.lax.rsqrt(deg)


def _dinv(part):
    return pl.pallas_call(
        _dinv_kernel,
        out_shape=jax.ShapeDtypeStruct((1, DEG_N), jnp.float32),
    )(part)


def _q1_kernel(x_ref, dinv_ref, o_ref):
    xb = x_ref[...] * dinv_ref[...]
    o_ref[...] = jnp.concatenate(
        [xb, jnp.zeros((xb.shape[0], 14), jnp.float32)], axis=1)


def _q1(x, dinv2d):
    blk = 10000
    return pl.pallas_call(
        _q1_kernel,
        grid=(N // blk,),
        in_specs=[
            pl.BlockSpec((blk, 18), lambda i: (i, 0)),
            pl.BlockSpec((blk, 1), lambda i: (i, 0)),
        ],
        out_specs=pl.BlockSpec((blk, 32), lambda i: (i, 0)),
        out_shape=jax.ShapeDtypeStruct((N, 32), jnp.float32),
    )(x, dinv2d)


def _mid_kernel(t_ref, q_ref, dinv_ref, w_ref, b_ref, o_ref):
    m = (t_ref[...] + q_ref[...]) * dinv_ref[...]
    h = jnp.maximum(jnp.dot(m, w_ref[...],
                            preferred_element_type=jnp.float32) + b_ref[...],
                    0.0)
    o_ref[...] = h * dinv_ref[...]


def _mid(t1, q1, dinv2d, W1p, b1):
    blk = 10000
    return pl.pallas_call(
        _mid_kernel,
        grid=(N // blk,),
        in_specs=[
            pl.BlockSpec((blk, 32), lambda i: (i, 0)),
            pl.BlockSpec((blk, 32), lambda i: (i, 0)),
            pl.BlockSpec((blk, 1), lambda i: (i, 0)),
            pl.BlockSpec((32, 32), lambda i: (0, 0)),
            pl.BlockSpec((1, 32), lambda i: (0, 0)),
        ],
        out_specs=pl.BlockSpec((blk, 32), lambda i: (i, 0)),
        out_shape=jax.ShapeDtypeStruct((N, 32), jnp.float32),
    )(t1, q1, dinv2d, W1p, b1.reshape(1, 32))


def _final_kernel(t_ref, q_ref, dinv_ref, w_ref, b_ref, wfc_ref, bfc_ref,
                  o_ref):
    m = (t_ref[...] + q_ref[...]) * dinv_ref[...]
    h = jnp.maximum(jnp.dot(m, w_ref[...],
                            preferred_element_type=jnp.float32) + b_ref[...],
                    0.0)
    logits = jnp.dot(h, wfc_ref[...],
                     preferred_element_type=jnp.float32) + bfc_ref[...]
    mx = jnp.max(logits, axis=1, keepdims=True)
    z = logits - mx
    lse = jnp.log(jnp.sum(jnp.exp(z), axis=1, keepdims=True))
    o_ref[...] = z - lse


def _final(t2, q2, dinv2d, W2, b2, Wfc, bfc):
    blk = 10000
    return pl.pallas_call(
        _final_kernel,
        grid=(N // blk,),
        in_specs=[
            pl.BlockSpec((blk, 32), lambda i: (i, 0)),
            pl.BlockSpec((blk, 32), lambda i: (i, 0)),
            pl.BlockSpec((blk, 1), lambda i: (i, 0)),
            pl.BlockSpec((32, 64), lambda i: (0, 0)),
            pl.BlockSpec((1, 64), lambda i: (0, 0)),
            pl.BlockSpec((64, 2), lambda i: (0, 0)),
            pl.BlockSpec((1, 2), lambda i: (0, 0)),
        ],
        out_specs=pl.BlockSpec((blk, 2), lambda i: (i, 0)),
        out_shape=jax.ShapeDtypeStruct((N, 2), jnp.float32),
    )(t2, q2, dinv2d, W2, b2.reshape(1, 64), Wfc, bfc.reshape(1, 2))


def kernel(x, edge_index, W1, b1, W2, b2, Wfc, bfc):
    src = edge_index[0].astype(jnp.int32)
    dst = edge_index[1].astype(jnp.int32)
    pad = E_PAD - E
    pad_i = jnp.arange(pad, dtype=jnp.int32)
    srcp = jnp.concatenate([src, pad_i % 128])
    dstp = jnp.concatenate([dst, N + (pad_i % 160)])
    idx1 = srcp[None, :] * 2 + jnp.arange(2, dtype=jnp.int32)[:, None]
    W1p = jnp.concatenate([W1, jnp.zeros((14, 32), jnp.float32)], axis=0)

    dummy = jnp.zeros((BLKW, 16), jnp.float32)
    dummyd = jnp.zeros((BLKW,), jnp.float32)

    part = _deg_kernel(dstp, dummyd).reshape(2, DEG_N)
    dinv2d = _dinv(part).reshape(DEG_N, 1)[:N]

    q1 = _q1(x, dinv2d)
    t1 = _agg32(q1.reshape(2 * N, 16), idx1, dstp, dummy)
    q2 = _mid(t1, q1, dinv2d, W1p, b1)
    t2 = _agg32(q2.reshape(2 * N, 16), idx1, dstp, dummy)
    return _final(t2, q2, dinv2d, W2, b2, Wfc, bfc)
